# baseline (speedup 1.0000x reference)
import functools

import jax
import jax.numpy as jnp
from jax import lax
from jax.experimental import pallas as pl
from jax.experimental.pallas import tpu as pltpu
from jax.experimental.pallas import tpu_sc as plsc

NUM_EMB = 1_000_000
D = 64
B_TOT = 4096 * 200
L = 16
NC, NS = 2, 16
NW = NC * NS
SUB = 128
KSUB = 2
CHUNK = SUB * KSUB
ROWS_W = B_TOT // NW
NSUB_W = ROWS_W // SUB
NCHUNK = ROWS_W // CHUNK
NSUPER = NCHUNK // 2
EPS = 1e-5
NVEC = D // L


_GDN = lax.GatherDimensionNumbers(
    offset_dims=(), collapsed_slice_dims=(0,), start_index_map=(0,))


def _lane_total(v, last):
    return lax.gather(plsc.cumsum(v), last, _GDN, slice_sizes=(1,),
                      mode=lax.GatherScatterMode.PROMISE_IN_BOUNDS)


def _ln_rows(rows_v, outb_v, buf, w_regs, b_regs):
    last = jnp.full((L, 1), L - 1, jnp.int32)
    for j in range(KSUB):
        def body(rr, _, j=j):
            a = [rows_v[buf, j, rr, pl.ds(c * L, L)] for c in range(NVEC)]
            s = (a[0] + a[1]) + (a[2] + a[3])
            q = (a[0] * a[0] + a[1] * a[1]) + (a[2] * a[2] + a[3] * a[3])
            s1 = _lane_total(s, last)
            s2 = _lane_total(q, last)
            mean = s1 * (1.0 / D)
            var = s2 * (1.0 / D) - mean * mean + EPS
            bits = lax.bitcast_convert_type(var, jnp.int32)
            y = lax.bitcast_convert_type(
                0x5F3759DF - lax.shift_right_logical(bits, 1), jnp.float32)
            h = 0.5 * var
            y = y * (1.5 - h * y * y)
            y = y * (1.5 - h * y * y)
            for c in range(NVEC):
                outb_v[buf, j, rr, pl.ds(c * L, L)] = (
                    (a[c] - mean) * y * w_regs[c] + b_regs[c])
            return 0
        lax.fori_loop(0, SUB, body, 0)


def _make_kernel():
    mesh = plsc.VectorSubcoreMesh(
        core_axis_name="c", subcore_axis_name="s",
        num_cores=NC, num_subcores=NS)

    @functools.partial(
        pl.kernel, mesh=mesh,
        compiler_params=pltpu.CompilerParams(
            needs_layout_passes=False, use_tc_tiling_on_sc=False),
        out_type=jax.ShapeDtypeStruct((B_TOT // SUB, SUB, D), jnp.float32),
        scratch_types=[
            pltpu.VMEM((NSUB_W, SUB), jnp.int32),
            pltpu.VMEM((2, KSUB, SUB, D), jnp.float32),
            pltpu.VMEM((2, KSUB, SUB, D), jnp.float32),
            pltpu.VMEM((D,), jnp.float32),
            pltpu.VMEM((D,), jnp.float32),
            pltpu.SemaphoreType.DMA,
            pltpu.SemaphoreType.DMA,
            pltpu.SemaphoreType.DMA,
            pltpu.SemaphoreType.DMA,
        ],
    )
    def k(x_hbm, table_hbm, w_hbm, bias_hbm, out_hbm,
          idx_v, rows_v, outb_v, w_v, b_v, sg0, sg1, sw0, sw1):
        wid = lax.axis_index("s") * NC + lax.axis_index("c")
        sub0 = wid * NSUB_W

        pltpu.sync_copy(w_hbm, w_v)
        pltpu.sync_copy(bias_hbm, b_v)
        pltpu.sync_copy(x_hbm.at[pl.ds(sub0, NSUB_W)], idx_v)

        sems_g = (sg0, sg1)
        sems_w = (sw0, sw1)

        def gather(buf, g, sem, wait):
            for j in range(KSUB):
                d = pltpu.make_async_copy(
                    table_hbm.at[idx_v.at[g * KSUB + j]],
                    rows_v.at[buf, j], sem)
                d.wait() if wait else d.start()

        def write(buf, g, sem, wait):
            for j in range(KSUB):
                d = pltpu.make_async_copy(
                    outb_v.at[buf, j],
                    out_hbm.at[sub0 + g * KSUB + j], sem)
                d.wait() if wait else d.start()

        w_regs = [w_v[pl.ds(c * L, L)] for c in range(NVEC)]
        b_regs = [b_v[pl.ds(c * L, L)] for c in range(NVEC)]

        gather(0, 0, sg0, False)
        gather(1, 1, sg1, False)

        def super_body(sstep, _):
            for b in range(2):
                g = 2 * sstep + b
                gather(b, g, sems_g[b], True)

                @pl.when(sstep >= 1)
                def _():
                    write(b, g - 2, sems_w[b], True)

                _ln_rows(rows_v, outb_v, b, w_regs, b_regs)

                @pl.when(sstep < NSUPER - 1)
                def _():
                    gather(b, g + 2, sems_g[b], False)

                write(b, g, sems_w[b], False)
            return 0

        lax.fori_loop(0, NSUPER, super_body, 0)

        write(0, NCHUNK - 2, sw0, True)
        write(1, NCHUNK - 1, sw1, True)

    return k


_kernel = _make_kernel()


@jax.jit
def kernel(x, table, ln_weight, ln_bias):
    bsz, seq = x.shape
    x2 = x.astype(jnp.int32).reshape(B_TOT // SUB, SUB)
    out = _kernel(x2, table, ln_weight, ln_bias)
    return out.reshape(bsz, seq, D)
